# Initial kernel scaffold; baseline (speedup 1.0000x reference)
import jax
import jax.numpy as jnp
from jax.experimental import pallas as pl
from jax.experimental.pallas import tpu as pltpu

_NF4_VALS = (
    -1.0, -0.6961928009986877, -0.5250730514526367, -0.39491748809814453,
    -0.28444138169288635, -0.18477343022823334, -0.09105003625154495, 0.0,
    0.07958029955625534, 0.16093020141124725, 0.24611230194568634,
    0.33791524171829224, 0.44070982933044434, 0.5626170039176941,
    0.7229568362236023, 1.0)

_QBLOCK = 64
_LORA_SCALE = 2.0


def _prep_kernel(w_idx_ref, scales_ref, lora_a_ref, lora_b_ref, w_out_ref):
    idx = w_idx_ref[...]
    ot, d = idx.shape
    cb = jnp.broadcast_to(
        jnp.array(_NF4_VALS, dtype=jnp.float32)[None, :], (ot, 16))
    deq = jnp.take_along_axis(cb, idx, axis=1)
    lane_blk = jax.lax.broadcasted_iota(jnp.int32, (ot, d), 1) // _QBLOCK
    sc = jnp.take_along_axis(scales_ref[...], lane_blk, axis=1)
    lora = jax.lax.dot_general(
        lora_b_ref[...], lora_a_ref[...],
        (((1,), (0,)), ((), ())), preferred_element_type=jnp.float32)
    w_out_ref[...] = (deq * sc + _LORA_SCALE * lora).astype(jnp.bfloat16)


def _matmul_kernel(x_ref, w_ref, y_ref):
    xb = x_ref[...].astype(jnp.bfloat16)
    y_ref[...] = jax.lax.dot_general(
        xb, w_ref[...], (((1,), (1,)), ((), ())),
        preferred_element_type=jnp.float32)


def kernel(x, w_idx, scales, lora_a, lora_b):
    b, s, d = x.shape
    o = w_idx.shape[0]
    m = b * s
    x2 = x.reshape(m, d)

    w_eff = pl.pallas_call(
        _prep_kernel,
        grid=(2,),
        in_specs=[
            pl.BlockSpec((o // 2, d), lambda i: (i, 0)),
            pl.BlockSpec((o // 2, scales.shape[1]), lambda i: (i, 0)),
            pl.BlockSpec(lora_a.shape, lambda i: (0, 0)),
            pl.BlockSpec((o // 2, lora_b.shape[1]), lambda i: (i, 0)),
        ],
        out_specs=pl.BlockSpec((o // 2, d), lambda i: (i, 0)),
        out_shape=jax.ShapeDtypeStruct((o, d), jnp.bfloat16),
        compiler_params=pltpu.CompilerParams(
            dimension_semantics=("parallel",),
        ),
        name="nf4_dequant_lora_merge",
    )(w_idx, scales, lora_a, lora_b)

    mt = 1024
    y2 = pl.pallas_call(
        _matmul_kernel,
        grid=(m // mt,),
        in_specs=[
            pl.BlockSpec((mt, d), lambda i: (i, 0)),
            pl.BlockSpec((o, d), lambda i: (0, 0)),
        ],
        out_specs=pl.BlockSpec((mt, o), lambda i: (i, 0)),
        out_shape=jax.ShapeDtypeStruct((m, o), jnp.float32),
        compiler_params=pltpu.CompilerParams(
            dimension_semantics=("parallel",),
            vmem_limit_bytes=60 * 1024 * 1024,
        ),
        name="qlora_matmul",
    )(x2, w_eff)

    return y2.reshape(b, s, o)


# speedup vs baseline: 4.7003x; 4.7003x over previous
import jax
import jax.numpy as jnp
from jax.experimental import pallas as pl
from jax.experimental.pallas import tpu as pltpu

_NF4_VALS = (
    -1.0, -0.6961928009986877, -0.5250730514526367, -0.39491748809814453,
    -0.28444138169288635, -0.18477343022823334, -0.09105003625154495, 0.0,
    0.07958029955625534, 0.16093020141124725, 0.24611230194568634,
    0.33791524171829224, 0.44070982933044434, 0.5626170039176941,
    0.7229568362236023, 1.0)

_QBLOCK = 64
_LORA_SCALE = 2.0


def _prep_kernel(cb_ref, w_idx_ref, scales_ref, lora_a_ref, lora_b_ref,
                 w_out_ref):
    idx = w_idx_ref[...]
    ot, d = idx.shape
    cb = jnp.broadcast_to(cb_ref[...], (ot, 16))
    deq = jnp.take_along_axis(cb, idx, axis=1)
    lane_blk = jax.lax.broadcasted_iota(jnp.int32, (ot, d), 1) // _QBLOCK
    sc = jnp.take_along_axis(scales_ref[...], lane_blk, axis=1)
    lora = jax.lax.dot_general(
        lora_b_ref[...], lora_a_ref[...],
        (((1,), (0,)), ((), ())), preferred_element_type=jnp.float32)
    w_out_ref[...] = (deq * sc + _LORA_SCALE * lora).astype(jnp.bfloat16)


def _matmul_kernel(x_ref, w_ref, y_ref):
    xb = x_ref[...].astype(jnp.bfloat16)
    y_ref[...] = jax.lax.dot_general(
        xb, w_ref[...], (((1,), (1,)), ((), ())),
        preferred_element_type=jnp.float32)


def kernel(x, w_idx, scales, lora_a, lora_b):
    b, s, d = x.shape
    o = w_idx.shape[0]
    m = b * s
    x2 = x.reshape(m, d)

    w_eff = pl.pallas_call(
        _prep_kernel,
        grid=(2,),
        in_specs=[
            pl.BlockSpec((1, 16), lambda i: (0, 0)),
            pl.BlockSpec((o // 2, d), lambda i: (i, 0)),
            pl.BlockSpec((o // 2, scales.shape[1]), lambda i: (i, 0)),
            pl.BlockSpec(lora_a.shape, lambda i: (0, 0)),
            pl.BlockSpec((o // 2, lora_b.shape[1]), lambda i: (i, 0)),
        ],
        out_specs=pl.BlockSpec((o // 2, d), lambda i: (i, 0)),
        out_shape=jax.ShapeDtypeStruct((o, d), jnp.bfloat16),
        compiler_params=pltpu.CompilerParams(
            dimension_semantics=("parallel",),
        ),
        name="nf4_dequant_lora_merge",
    )(jnp.array(_NF4_VALS, dtype=jnp.float32).reshape(1, 16),
      w_idx, scales, lora_a, lora_b)

    mt = 1024
    y2 = pl.pallas_call(
        _matmul_kernel,
        grid=(m // mt,),
        in_specs=[
            pl.BlockSpec((mt, d), lambda i: (i, 0)),
            pl.BlockSpec((o, d), lambda i: (0, 0)),
        ],
        out_specs=pl.BlockSpec((mt, o), lambda i: (i, 0)),
        out_shape=jax.ShapeDtypeStruct((m, o), jnp.float32),
        compiler_params=pltpu.CompilerParams(
            dimension_semantics=("parallel",),
            vmem_limit_bytes=60 * 1024 * 1024,
        ),
        name="qlora_matmul",
    )(x2, w_eff)

    return y2.reshape(b, s, o)
